# Initial kernel scaffold; baseline (speedup 1.0000x reference)
import jax
import jax.numpy as jnp
from jax.experimental import pallas as pl


def kernel(H, edge_index, Wx, bx, Wh, bh, wc, bg, ln_g, ln_b, W1, b1, W2, b2):
    raise NotImplementedError("write your pallas kernel here")


# speedup vs baseline: 1.7474x; 1.7474x over previous
import functools

import jax
import jax.numpy as jnp
from jax import lax
from jax.experimental import pallas as pl
from jax.experimental.pallas import tpu as pltpu
from jax.experimental.pallas import tpu_sc as plsc

N = 10000
E = 160000
SEQ = 12
LIN = 256
CONV = 256
CH = 128
HID = (CONV + CH) // 2

NP_ = 10112
EP = 163840
CH_E = 128
EPT = EP // 16
EPW = EP // 32
ROWS_T = NP_ // 16

_mesh = plsc.VectorSubcoreMesh(core_axis_name="c", subcore_axis_name="s")


@functools.partial(
    pl.kernel,
    out_type=jax.ShapeDtypeStruct((2, NP_, 128), jnp.float32),
    mesh=_mesh,
    scratch_types=[
        pltpu.VMEM((CH_E,), jnp.int32),
        pltpu.VMEM((CH_E, 128), jnp.float32),
        pltpu.VMEM((ROWS_T // 8, 128), jnp.float32),
        pltpu.VMEM_SHARED((NP_, 128), jnp.float32),
    ],
)
def _deg_kernel(degidx_hbm, out_hbm, idx_v, ones_v, zbuf, acc):
    c = lax.axis_index("c")
    s = lax.axis_index("s")
    w = c * 16 + s
    one16 = jnp.ones((16,), jnp.float32)
    zero16 = jnp.zeros((16,), jnp.float32)

    @pl.loop(0, ROWS_T // 8)
    def _(i):
        for j in range(8):
            zbuf[i, pl.ds(j * 16, 16)] = zero16

    @pl.loop(0, CH_E)
    def _(i):
        for j in range(8):
            ones_v[i, pl.ds(j * 16, 16)] = one16

    for q in range(8):
        pltpu.sync_copy(
            zbuf,
            acc.at[pl.ds(s * ROWS_T + q * (ROWS_T // 8), ROWS_T // 8)])
    plsc.subcore_barrier()

    @pl.loop(0, EPW // CH_E)
    def _(k):
        base = w * EPW + k * CH_E
        pltpu.sync_copy(degidx_hbm.at[pl.ds(base, CH_E)], idx_v)
        pltpu.sync_copy(ones_v, acc.at[idx_v], add=True)

    plsc.subcore_barrier()
    pltpu.sync_copy(acc.at[pl.ds(s * ROWS_T, ROWS_T)],
                    out_hbm.at[c, pl.ds(s * ROWS_T, ROWS_T)])


@functools.partial(
    pl.kernel,
    out_type=jax.ShapeDtypeStruct((EP, 16), jnp.float32),
    mesh=_mesh,
    scratch_types=[
        pltpu.VMEM((CH_E,), jnp.int32),
        pltpu.VMEM((CH_E,), jnp.int32),
        pltpu.VMEM((CH_E, 128), jnp.float32),
        pltpu.VMEM((CH_E, 128), jnp.float32),
        pltpu.VMEM((CH_E, 16), jnp.float32),
        pltpu.SemaphoreType.DMA,
    ],
)
def _norm_kernel(dis_hbm, src_hbm, dst_hbm, out_hbm, sv, dv, av, bv, nv, sem):
    c = lax.axis_index("c")
    s = lax.axis_index("s")
    w = c * 16 + s

    @pl.loop(0, EPW // CH_E)
    def _(k):
        base = w * EPW + k * CH_E
        pltpu.sync_copy(src_hbm.at[pl.ds(base, CH_E)], sv)
        pltpu.sync_copy(dst_hbm.at[pl.ds(base, CH_E)], dv)
        pltpu.async_copy(dis_hbm.at[sv], av, sem).wait()
        pltpu.async_copy(dis_hbm.at[dv], bv, sem).wait()

        @pl.loop(0, CH_E)
        def _(e):
            nv[e] = -(av[e, pl.ds(0, 16)] * bv[e, pl.ds(0, 16)])

        pltpu.sync_copy(nv, out_hbm.at[pl.ds(base, CH_E)])


def _make_lap_kernel(T):
    @functools.partial(
        pl.kernel,
        out_type=jax.ShapeDtypeStruct((T, NP_, 256), jnp.float32),
        mesh=_mesh,
        scratch_types=[
            pltpu.VMEM((CH_E,), jnp.int32),
            pltpu.VMEM((CH_E,), jnp.int32),
            pltpu.VMEM((CH_E, 16), jnp.float32),
            pltpu.VMEM((CH_E, 128), jnp.float32),
            pltpu.VMEM((ROWS_T // 8, 128), jnp.float32),
            pltpu.VMEM_SHARED((NP_, 128), jnp.float32),
            pltpu.SemaphoreType.DMA,
        ],
    )
    def lap_kernel(tab_hbm, gidx_hbm, sidx_hbm, nrm_hbm, out_hbm,
                   gi_v, si_v, nm_v, rows, zbuf, acc, sem):
        c = lax.axis_index("c")
        s = lax.axis_index("s")
        zero16 = jnp.zeros((16,), jnp.float32)

        @pl.loop(0, ROWS_T // 8)
        def _(i):
            for j in range(8):
                zbuf[i, pl.ds(j * 16, 16)] = zero16

        def zero_slab():
            for q in range(8):
                pltpu.sync_copy(
                    zbuf,
                    acc.at[pl.ds(s * ROWS_T + q * (ROWS_T // 8),
                                 ROWS_T // 8)])

        zero_slab()
        plsc.subcore_barrier()

        @pl.loop(0, T)
        def _(t):
            @pl.loop(0, EPT // CH_E)
            def _(k):
                base = s * EPT + k * CH_E
                pltpu.sync_copy(gidx_hbm.at[t, c, pl.ds(base, CH_E)], gi_v)
                pltpu.sync_copy(sidx_hbm.at[pl.ds(base, CH_E)], si_v)
                pltpu.sync_copy(nrm_hbm.at[pl.ds(base, CH_E)], nm_v)
                pltpu.async_copy(tab_hbm.at[gi_v], rows, sem).wait()

                @pl.loop(0, CH_E)
                def _(e):
                    nrow = nm_v[e]
                    for j in range(8):
                        rows[e, pl.ds(j * 16, 16)] = (
                            rows[e, pl.ds(j * 16, 16)] * nrow)

                pltpu.sync_copy(rows, acc.at[si_v], add=True)

            plsc.subcore_barrier()
            pltpu.sync_copy(acc.at[pl.ds(s * ROWS_T, ROWS_T)],
                            out_hbm.at[t, pl.ds(s * ROWS_T, ROWS_T),
                                       pl.ds(c * 128, 128)])
            zero_slab()
            plsc.subcore_barrier()

    return lap_kernel


_lap12 = _make_lap_kernel(SEQ)
_lap1 = _make_lap_kernel(1)


def _dis_body(deg_ref, o_ref):
    d = jnp.sum(deg_ref[...], axis=(0, 2)) * (1.0 / 128.0)
    v = jnp.where(d > 0, 1.0 / jnp.sqrt(jnp.maximum(d, 1e-12)), 0.0)
    o_ref[...] = jnp.broadcast_to(v[:, None], (NP_, 128))


_BN = 1000


def _gates_body(t_ref, x_ref, lx1_ref, lx2_ref, h_ref, lh_ref, llh_ref,
                c_ref, w_ref, b_ref, wc_ref, hn_ref, cn_ref):
    X = jnp.concatenate(
        [x_ref[0], lx1_ref[0], lx2_ref[0], h_ref[...], lh_ref[0],
         llh_ref[0]], axis=-1)
    G = lax.dot_general(X, w_ref[...], (((1,), (0,)), ((), ())),
                        preferred_element_type=jnp.float32) + b_ref[...]
    cb = c_ref[...]
    gi = jax.nn.sigmoid(G[:, 0:256] + wc_ref[0:1, :] * cb)
    gf = jax.nn.sigmoid(G[:, 256:512] + wc_ref[1:2, :] * cb)
    gt = jnp.tanh(G[:, 512:768])
    cn = gf * cb + gi * gt
    go = jax.nn.sigmoid(G[:, 768:1024] + wc_ref[2:3, :] * cn)
    hn_ref[...] = go * jnp.tanh(cn)
    cn_ref[...] = cn


_gates_call = pl.pallas_call(
    _gates_body,
    grid_spec=pltpu.PrefetchScalarGridSpec(
        num_scalar_prefetch=1,
        grid=(N // _BN,),
        in_specs=[
            pl.BlockSpec((1, _BN, 256), lambda i, t: (t[0], i, 0)),
            pl.BlockSpec((1, _BN, 256), lambda i, t: (t[0], i, 0)),
            pl.BlockSpec((1, _BN, 256), lambda i, t: (t[0], i, 0)),
            pl.BlockSpec((_BN, 256), lambda i, t: (i, 0)),
            pl.BlockSpec((1, _BN, 256), lambda i, t: (0, i, 0)),
            pl.BlockSpec((1, _BN, 256), lambda i, t: (0, i, 0)),
            pl.BlockSpec((_BN, 256), lambda i, t: (i, 0)),
            pl.BlockSpec((1536, 1024), lambda i, t: (0, 0)),
            pl.BlockSpec((1, 1024), lambda i, t: (0, 0)),
            pl.BlockSpec((3, 256), lambda i, t: (0, 0)),
        ],
        out_specs=[
            pl.BlockSpec((_BN, 256), lambda i, t: (i, 0)),
            pl.BlockSpec((_BN, 256), lambda i, t: (i, 0)),
        ],
    ),
    out_shape=[
        jax.ShapeDtypeStruct((N, 256), jnp.float32),
        jax.ShapeDtypeStruct((N, 256), jnp.float32),
    ],
)


def _head_body(*refs):
    h_refs = refs[:SEQ]
    lng_ref, lnb_ref, w1_ref, b1_ref, w2_ref, b2_ref, o_ref = refs[SEQ:]
    ys = []
    for t in range(SEQ):
        x = jnp.tanh(h_refs[t][...])
        mu = jnp.mean(x, axis=-1, keepdims=True)
        xc = x - mu
        var = jnp.mean(xc * xc, axis=-1, keepdims=True)
        xn = xc / jnp.sqrt(var + 1e-5) * lng_ref[...] + lnb_ref[...]
        y = jnp.maximum(
            lax.dot_general(xn, w1_ref[...], (((1,), (0,)), ((), ())),
                            preferred_element_type=jnp.float32)
            + b1_ref[...], 0.0)
        y = lax.dot_general(y, w2_ref[...], (((1,), (0,)), ((), ())),
                            preferred_element_type=jnp.float32) + b2_ref[...]
        ys.append(jax.nn.sigmoid(y)[:, None, :])
    o_ref[...] = jnp.concatenate(ys, axis=1)


_HBN = 1000
_head_call = pl.pallas_call(
    _head_body,
    grid=(N // _HBN,),
    in_specs=(
        [pl.BlockSpec((_HBN, 256), lambda i: (i, 0)) for _ in range(SEQ)]
        + [
            pl.BlockSpec((1, 256), lambda i: (0, 0)),
            pl.BlockSpec((1, 256), lambda i: (0, 0)),
            pl.BlockSpec((256, HID), lambda i: (0, 0)),
            pl.BlockSpec((1, HID), lambda i: (0, 0)),
            pl.BlockSpec((HID, CH), lambda i: (0, 0)),
            pl.BlockSpec((1, CH), lambda i: (0, 0)),
        ]
    ),
    out_specs=pl.BlockSpec((_HBN, SEQ, CH), lambda i: (i, 0, 0)),
    out_shape=jax.ShapeDtypeStruct((N, SEQ, CH), jnp.float32),
)


def _stack_w(W):
    blocks = jnp.concatenate([W[:, 0] - W[:, 2], W[:, 1], 2.0 * W[:, 2]],
                             axis=1)
    return jnp.moveaxis(blocks, 0, 1).reshape(768, 1024)


def kernel(H, edge_index, Wx, bx, Wh, bh, wc, bg, ln_g, ln_b, W1, b1, W2, b2):
    src = edge_index[0]
    dst = edge_index[1]
    pad = EP - E
    zpad = jnp.zeros((pad,), jnp.int32)
    tpad = jnp.full((pad,), N, jnp.int32)
    srcp0 = jnp.concatenate([src, zpad])
    dstp0 = jnp.concatenate([dst, zpad])
    sidx = jnp.concatenate([dst, tpad])
    degidx = jnp.concatenate([src, tpad])

    hfo = jnp.arange(2, dtype=jnp.int32)[None, :, None]
    t_ar = jnp.arange(SEQ, dtype=jnp.int32)[:, None, None]
    base2 = (2 * srcp0)[None, None, :]
    gidxH = (2 * N) * t_ar + hfo + base2
    gidxL = (2 * NP_) * t_ar + hfo + base2
    idx0 = gidxH[:1]

    Wbig = jnp.concatenate([_stack_w(Wx), _stack_w(Wh)], axis=0)
    bbig = (bx + bh + bg).reshape(1, 1024)
    wc2 = wc

    degparts = _deg_kernel(degidx)
    dis = pl.pallas_call(
        _dis_body,
        out_shape=jax.ShapeDtypeStruct((NP_, 128), jnp.float32),
    )(degparts)
    norm = _norm_kernel(dis, srcp0, dstp0)

    Htab = H.reshape(SEQ * N * 2, 128)
    LX1 = _lap12(Htab, gidxH, sidx, norm)
    LX2 = _lap12(LX1.reshape(SEQ * NP_ * 2, 128), gidxL, sidx, norm)
    LX1r = LX1.reshape(SEQ, NP_, 256)
    LX2r = LX2.reshape(SEQ, NP_, 256)

    h = jnp.zeros((N, 256), jnp.float32)
    c = jnp.zeros((N, 256), jnp.float32)
    zlap = jnp.zeros((1, NP_, 256), jnp.float32)
    hs = []
    for t in range(SEQ):
        if t == 0:
            lh, llh = zlap, zlap
        else:
            lh4 = _lap1(h.reshape(N * 2, 128), idx0, sidx, norm)
            llh4 = _lap1(lh4.reshape(NP_ * 2, 128), gidxL[:1], sidx, norm)
            lh = lh4.reshape(1, NP_, 256)
            llh = llh4.reshape(1, NP_, 256)
        tt = jnp.full((1,), t, jnp.int32)
        h, c = _gates_call(tt, H, LX1r, LX2r, h, lh, llh, c, Wbig, bbig, wc2)
        hs.append(h)

    out = _head_call(*hs, ln_g.reshape(1, 256), ln_b.reshape(1, 256),
                     W1, b1.reshape(1, HID), W2, b2.reshape(1, CH))
    return jnp.swapaxes(out, 1, 2)


# speedup vs baseline: 2.6187x; 1.4986x over previous
import functools

import jax
import jax.numpy as jnp
from jax import lax
from jax.experimental import pallas as pl
from jax.experimental.pallas import tpu as pltpu
from jax.experimental.pallas import tpu_sc as plsc

N = 10000
E = 160000
SEQ = 12
LIN = 256
CONV = 256
CH = 128
HID = (CONV + CH) // 2

NP_ = 10112
EP = 163840
CH_E = 128
EPT = EP // 16
EPW = EP // 32
ROWS_T = NP_ // 16

_mesh = plsc.VectorSubcoreMesh(core_axis_name="c", subcore_axis_name="s")


@functools.partial(
    pl.kernel,
    out_type=jax.ShapeDtypeStruct((2, NP_, 128), jnp.float32),
    mesh=_mesh,
    scratch_types=[
        pltpu.VMEM((CH_E,), jnp.int32),
        pltpu.VMEM((CH_E, 128), jnp.float32),
        pltpu.VMEM((ROWS_T // 8, 128), jnp.float32),
        pltpu.VMEM_SHARED((NP_, 128), jnp.float32),
    ],
)
def _deg_kernel(degidx_hbm, out_hbm, idx_v, ones_v, zbuf, acc):
    c = lax.axis_index("c")
    s = lax.axis_index("s")
    w = c * 16 + s
    one16 = jnp.ones((16,), jnp.float32)
    zero16 = jnp.zeros((16,), jnp.float32)

    @pl.loop(0, ROWS_T // 8)
    def _(i):
        for j in range(8):
            zbuf[i, pl.ds(j * 16, 16)] = zero16

    @pl.loop(0, CH_E)
    def _(i):
        for j in range(8):
            ones_v[i, pl.ds(j * 16, 16)] = one16

    for q in range(8):
        pltpu.sync_copy(
            zbuf,
            acc.at[pl.ds(s * ROWS_T + q * (ROWS_T // 8), ROWS_T // 8)])
    plsc.subcore_barrier()

    @pl.loop(0, EPW // CH_E)
    def _(k):
        base = w * EPW + k * CH_E
        pltpu.sync_copy(degidx_hbm.at[pl.ds(base, CH_E)], idx_v)
        pltpu.sync_copy(ones_v, acc.at[idx_v], add=True)

    plsc.subcore_barrier()
    pltpu.sync_copy(acc.at[pl.ds(s * ROWS_T, ROWS_T)],
                    out_hbm.at[c, pl.ds(s * ROWS_T, ROWS_T)])


@functools.partial(
    pl.kernel,
    out_type=jax.ShapeDtypeStruct((EP, 16), jnp.float32),
    mesh=_mesh,
    scratch_types=[
        pltpu.VMEM((CH_E,), jnp.int32),
        pltpu.VMEM((CH_E,), jnp.int32),
        pltpu.VMEM((CH_E, 128), jnp.float32),
        pltpu.VMEM((CH_E, 128), jnp.float32),
        pltpu.VMEM((CH_E, 16), jnp.float32),
        pltpu.SemaphoreType.DMA,
    ],
)
def _norm_kernel(dis_hbm, src_hbm, dst_hbm, out_hbm, sv, dv, av, bv, nv, sem):
    c = lax.axis_index("c")
    s = lax.axis_index("s")
    w = c * 16 + s

    @pl.loop(0, EPW // CH_E)
    def _(k):
        base = w * EPW + k * CH_E
        pltpu.sync_copy(src_hbm.at[pl.ds(base, CH_E)], sv)
        pltpu.sync_copy(dst_hbm.at[pl.ds(base, CH_E)], dv)
        pltpu.async_copy(dis_hbm.at[sv], av, sem).wait()
        pltpu.async_copy(dis_hbm.at[dv], bv, sem).wait()

        @pl.loop(0, CH_E)
        def _(e):
            nv[e] = -(av[e, pl.ds(0, 16)] * bv[e, pl.ds(0, 16)])

        pltpu.sync_copy(nv, out_hbm.at[pl.ds(base, CH_E)])


NCH = EPT // CH_E


def _make_lap_kernel(T):
    @functools.partial(
        pl.kernel,
        out_type=jax.ShapeDtypeStruct((T, NP_, 256), jnp.float32),
        mesh=_mesh,
        scratch_types=[
            pltpu.VMEM((2, CH_E), jnp.int32),
            pltpu.VMEM((2, CH_E), jnp.int32),
            pltpu.VMEM((2, CH_E * 16), jnp.float32),
            pltpu.VMEM((2, CH_E, 128), jnp.float32),
            pltpu.VMEM((ROWS_T // 8, 128), jnp.float32),
            pltpu.VMEM_SHARED((NP_, 128), jnp.float32),
            pltpu.SemaphoreType.DMA,
            pltpu.SemaphoreType.DMA,
            pltpu.SemaphoreType.DMA,
            pltpu.SemaphoreType.DMA,
        ],
    )
    def lap_kernel(tab_hbm, gidx_hbm, sidx_hbm, nrm_hbm, out_hbm,
                   gi_c, si_c, nm_c, rows, zbuf, acc,
                   sg0, sg1, si0, si1):
        c = lax.axis_index("c")
        s = lax.axis_index("s")
        zero16 = jnp.zeros((16,), jnp.float32)
        sgs = (sg0, sg1)
        sis = (si0, si1)

        @pl.loop(0, ROWS_T // 8)
        def _(i):
            for j in range(8):
                zbuf[i, pl.ds(j * 16, 16)] = zero16

        def zero_slab():
            for q in range(8):
                pltpu.sync_copy(
                    zbuf,
                    acc.at[pl.ds(s * ROWS_T + q * (ROWS_T // 8),
                                 ROWS_T // 8)])

        zero_slab()
        plsc.subcore_barrier()

        @pl.loop(0, T)
        def _(t):
            def chunk_refs(k):
                base = s * EPT + k * CH_E
                return (gidx_hbm.at[t, c, pl.ds(base, CH_E)],
                        sidx_hbm.at[pl.ds(base, CH_E)],
                        nrm_hbm.at[pl.ds(base * 16, CH_E * 16)])

            def fire_loads(k, b):
                g_src, s_src, n_src = chunk_refs(k)
                pltpu.async_copy(g_src, gi_c.at[b], sis[b])
                pltpu.async_copy(s_src, si_c.at[b], sis[b])
                pltpu.async_copy(n_src, nm_c.at[b], sis[b])

            def wait_loads(k, b):
                g_src, s_src, n_src = chunk_refs(k)
                pltpu.make_async_copy(g_src, gi_c.at[b], sis[b]).wait()
                pltpu.make_async_copy(s_src, si_c.at[b], sis[b]).wait()
                pltpu.make_async_copy(n_src, nm_c.at[b], sis[b]).wait()

            def fire_gather(b):
                pltpu.async_copy(tab_hbm.at[gi_c.at[b]], rows.at[b], sgs[b])

            def wait_gather(b):
                pltpu.make_async_copy(
                    tab_hbm.at[gi_c.at[b]], rows.at[b], sgs[b]).wait()

            g_src, s_src, n_src = chunk_refs(0)
            pltpu.sync_copy(g_src, gi_c.at[0])
            pltpu.sync_copy(s_src, si_c.at[0])
            pltpu.sync_copy(n_src, nm_c.at[0])
            fire_gather(0)
            fire_loads(1, 1)

            @pl.loop(0, NCH, step=2)
            def _(k0):
                for b in range(2):
                    k = k0 + b
                    nb = 1 - b

                    @pl.when(k + 1 < NCH)
                    def _():
                        wait_loads(k + 1, nb)

                    wait_gather(b)

                    @pl.when(k + 1 < NCH)
                    def _():
                        fire_gather(nb)

                    @pl.loop(0, CH_E)
                    def _(e):
                        nrow = nm_c[b, pl.ds(e * 16, 16)]
                        for j in range(8):
                            rows[b, e, pl.ds(j * 16, 16)] = (
                                rows[b, e, pl.ds(j * 16, 16)] * nrow)

                    pltpu.sync_copy(rows.at[b], acc.at[si_c.at[b]],
                                    add=True)

                    @pl.when(k + 2 < NCH)
                    def _():
                        fire_loads(k + 2, b)

            plsc.subcore_barrier()
            pltpu.sync_copy(acc.at[pl.ds(s * ROWS_T, ROWS_T)],
                            out_hbm.at[t, pl.ds(s * ROWS_T, ROWS_T),
                                       pl.ds(c * 128, 128)])
            zero_slab()
            plsc.subcore_barrier()

    return lap_kernel


_lap12 = _make_lap_kernel(SEQ)
_lap1 = _make_lap_kernel(1)


def _dis_body(deg_ref, o_ref):
    d = jnp.sum(deg_ref[...], axis=(0, 2)) * (1.0 / 128.0)
    v = jnp.where(d > 0, 1.0 / jnp.sqrt(jnp.maximum(d, 1e-12)), 0.0)
    o_ref[...] = jnp.broadcast_to(v[:, None], (NP_, 128))


_BN = 1000


def _gates_body(t_ref, x_ref, lx1_ref, lx2_ref, h_ref, lh_ref, llh_ref,
                c_ref, w_ref, b_ref, wc_ref, hn_ref, cn_ref):
    X = jnp.concatenate(
        [x_ref[0], lx1_ref[0], lx2_ref[0], h_ref[...], lh_ref[0],
         llh_ref[0]], axis=-1)
    G = lax.dot_general(X, w_ref[...], (((1,), (0,)), ((), ())),
                        preferred_element_type=jnp.float32) + b_ref[...]
    cb = c_ref[...]
    gi = jax.nn.sigmoid(G[:, 0:256] + wc_ref[0:1, :] * cb)
    gf = jax.nn.sigmoid(G[:, 256:512] + wc_ref[1:2, :] * cb)
    gt = jnp.tanh(G[:, 512:768])
    cn = gf * cb + gi * gt
    go = jax.nn.sigmoid(G[:, 768:1024] + wc_ref[2:3, :] * cn)
    hn_ref[...] = go * jnp.tanh(cn)
    cn_ref[...] = cn


_gates_call = pl.pallas_call(
    _gates_body,
    grid_spec=pltpu.PrefetchScalarGridSpec(
        num_scalar_prefetch=1,
        grid=(N // _BN,),
        in_specs=[
            pl.BlockSpec((1, _BN, 256), lambda i, t: (t[0], i, 0)),
            pl.BlockSpec((1, _BN, 256), lambda i, t: (t[0], i, 0)),
            pl.BlockSpec((1, _BN, 256), lambda i, t: (t[0], i, 0)),
            pl.BlockSpec((_BN, 256), lambda i, t: (i, 0)),
            pl.BlockSpec((1, _BN, 256), lambda i, t: (0, i, 0)),
            pl.BlockSpec((1, _BN, 256), lambda i, t: (0, i, 0)),
            pl.BlockSpec((_BN, 256), lambda i, t: (i, 0)),
            pl.BlockSpec((1536, 1024), lambda i, t: (0, 0)),
            pl.BlockSpec((1, 1024), lambda i, t: (0, 0)),
            pl.BlockSpec((3, 256), lambda i, t: (0, 0)),
        ],
        out_specs=[
            pl.BlockSpec((_BN, 256), lambda i, t: (i, 0)),
            pl.BlockSpec((_BN, 256), lambda i, t: (i, 0)),
        ],
    ),
    out_shape=[
        jax.ShapeDtypeStruct((N, 256), jnp.float32),
        jax.ShapeDtypeStruct((N, 256), jnp.float32),
    ],
)


def _head_body(*refs):
    h_refs = refs[:SEQ]
    lng_ref, lnb_ref, w1_ref, b1_ref, w2_ref, b2_ref, o_ref = refs[SEQ:]
    ys = []
    for t in range(SEQ):
        x = jnp.tanh(h_refs[t][...])
        mu = jnp.mean(x, axis=-1, keepdims=True)
        xc = x - mu
        var = jnp.mean(xc * xc, axis=-1, keepdims=True)
        xn = xc / jnp.sqrt(var + 1e-5) * lng_ref[...] + lnb_ref[...]
        y = jnp.maximum(
            lax.dot_general(xn, w1_ref[...], (((1,), (0,)), ((), ())),
                            preferred_element_type=jnp.float32)
            + b1_ref[...], 0.0)
        y = lax.dot_general(y, w2_ref[...], (((1,), (0,)), ((), ())),
                            preferred_element_type=jnp.float32) + b2_ref[...]
        ys.append(jax.nn.sigmoid(y)[:, None, :])
    o_ref[...] = jnp.concatenate(ys, axis=1)


_HBN = 1000
_head_call = pl.pallas_call(
    _head_body,
    grid=(N // _HBN,),
    in_specs=(
        [pl.BlockSpec((_HBN, 256), lambda i: (i, 0)) for _ in range(SEQ)]
        + [
            pl.BlockSpec((1, 256), lambda i: (0, 0)),
            pl.BlockSpec((1, 256), lambda i: (0, 0)),
            pl.BlockSpec((256, HID), lambda i: (0, 0)),
            pl.BlockSpec((1, HID), lambda i: (0, 0)),
            pl.BlockSpec((HID, CH), lambda i: (0, 0)),
            pl.BlockSpec((1, CH), lambda i: (0, 0)),
        ]
    ),
    out_specs=pl.BlockSpec((_HBN, SEQ, CH), lambda i: (i, 0, 0)),
    out_shape=jax.ShapeDtypeStruct((N, SEQ, CH), jnp.float32),
)


def _stack_w(W):
    blocks = jnp.concatenate([W[:, 0] - W[:, 2], W[:, 1], 2.0 * W[:, 2]],
                             axis=1)
    return jnp.moveaxis(blocks, 0, 1).reshape(768, 1024)


def kernel(H, edge_index, Wx, bx, Wh, bh, wc, bg, ln_g, ln_b, W1, b1, W2, b2):
    src = edge_index[0]
    dst = edge_index[1]
    pad = EP - E
    zpad = jnp.zeros((pad,), jnp.int32)
    tpad = jnp.full((pad,), N, jnp.int32)
    srcp0 = jnp.concatenate([src, zpad])
    dstp0 = jnp.concatenate([dst, zpad])
    sidx = jnp.concatenate([dst, tpad])
    degidx = jnp.concatenate([src, tpad])

    hfo = jnp.arange(2, dtype=jnp.int32)[None, :, None]
    t_ar = jnp.arange(SEQ, dtype=jnp.int32)[:, None, None]
    base2 = (2 * srcp0)[None, None, :]
    gidxH = (2 * N) * t_ar + hfo + base2
    gidxL = (2 * NP_) * t_ar + hfo + base2
    idx0 = gidxH[:1]

    Wbig = jnp.concatenate([_stack_w(Wx), _stack_w(Wh)], axis=0)
    bbig = (bx + bh + bg).reshape(1, 1024)
    wc2 = wc

    degparts = _deg_kernel(degidx)
    dis = pl.pallas_call(
        _dis_body,
        out_shape=jax.ShapeDtypeStruct((NP_, 128), jnp.float32),
    )(degparts)
    norm = _norm_kernel(dis, srcp0, dstp0).reshape(EP * 16)

    Htab = H.reshape(SEQ * N * 2, 128)
    LX1 = _lap12(Htab, gidxH, sidx, norm)
    LX2 = _lap12(LX1.reshape(SEQ * NP_ * 2, 128), gidxL, sidx, norm)
    LX1r = LX1.reshape(SEQ, NP_, 256)
    LX2r = LX2.reshape(SEQ, NP_, 256)

    h = jnp.zeros((N, 256), jnp.float32)
    c = jnp.zeros((N, 256), jnp.float32)
    zlap = jnp.zeros((1, NP_, 256), jnp.float32)
    hs = []
    for t in range(SEQ):
        if t == 0:
            lh, llh = zlap, zlap
        else:
            lh4 = _lap1(h.reshape(N * 2, 128), idx0, sidx, norm)
            llh4 = _lap1(lh4.reshape(NP_ * 2, 128), gidxL[:1], sidx, norm)
            lh = lh4.reshape(1, NP_, 256)
            llh = llh4.reshape(1, NP_, 256)
        tt = jnp.full((1,), t, jnp.int32)
        h, c = _gates_call(tt, H, LX1r, LX2r, h, lh, llh, c, Wbig, bbig, wc2)
        hs.append(h)

    out = _head_call(*hs, ln_g.reshape(1, 256), ln_b.reshape(1, 256),
                     W1, b1.reshape(1, HID), W2, b2.reshape(1, CH))
    return jnp.swapaxes(out, 1, 2)


# speedup vs baseline: 2.8398x; 1.0844x over previous
import functools

import jax
import jax.numpy as jnp
from jax import lax
from jax.experimental import pallas as pl
from jax.experimental.pallas import tpu as pltpu
from jax.experimental.pallas import tpu_sc as plsc

N = 10000
E = 160000
SEQ = 12
LIN = 256
CONV = 256
CH = 128
HID = (CONV + CH) // 2

NP_ = 10112
EP = 163840
CH_E = 128
EPT = EP // 16
EPW = EP // 32
ROWS_T = NP_ // 16

_mesh = plsc.VectorSubcoreMesh(core_axis_name="c", subcore_axis_name="s")


@functools.partial(
    pl.kernel,
    out_type=jax.ShapeDtypeStruct((2, NP_, 128), jnp.float32),
    mesh=_mesh,
    scratch_types=[
        pltpu.VMEM((CH_E,), jnp.int32),
        pltpu.VMEM((CH_E, 128), jnp.float32),
        pltpu.VMEM((ROWS_T // 8, 128), jnp.float32),
        pltpu.VMEM_SHARED((NP_, 128), jnp.float32),
    ],
)
def _deg_kernel(degidx_hbm, out_hbm, idx_v, ones_v, zbuf, acc):
    c = lax.axis_index("c")
    s = lax.axis_index("s")
    w = c * 16 + s
    one16 = jnp.ones((16,), jnp.float32)
    zero16 = jnp.zeros((16,), jnp.float32)

    @pl.loop(0, ROWS_T // 8)
    def _(i):
        for j in range(8):
            zbuf[i, pl.ds(j * 16, 16)] = zero16

    @pl.loop(0, CH_E)
    def _(i):
        for j in range(8):
            ones_v[i, pl.ds(j * 16, 16)] = one16

    for q in range(8):
        pltpu.sync_copy(
            zbuf,
            acc.at[pl.ds(s * ROWS_T + q * (ROWS_T // 8), ROWS_T // 8)])
    plsc.subcore_barrier()

    @pl.loop(0, EPW // CH_E)
    def _(k):
        base = w * EPW + k * CH_E
        pltpu.sync_copy(degidx_hbm.at[pl.ds(base, CH_E)], idx_v)
        pltpu.sync_copy(ones_v, acc.at[idx_v], add=True)

    plsc.subcore_barrier()
    pltpu.sync_copy(acc.at[pl.ds(s * ROWS_T, ROWS_T)],
                    out_hbm.at[c, pl.ds(s * ROWS_T, ROWS_T)])


@functools.partial(
    pl.kernel,
    out_type=jax.ShapeDtypeStruct((EP, 16), jnp.float32),
    mesh=_mesh,
    scratch_types=[
        pltpu.VMEM((CH_E,), jnp.int32),
        pltpu.VMEM((CH_E,), jnp.int32),
        pltpu.VMEM((CH_E, 128), jnp.float32),
        pltpu.VMEM((CH_E, 128), jnp.float32),
        pltpu.VMEM((CH_E, 16), jnp.float32),
        pltpu.SemaphoreType.DMA,
    ],
)
def _norm_kernel(dis_hbm, src_hbm, dst_hbm, out_hbm, sv, dv, av, bv, nv, sem):
    c = lax.axis_index("c")
    s = lax.axis_index("s")
    w = c * 16 + s

    @pl.loop(0, EPW // CH_E)
    def _(k):
        base = w * EPW + k * CH_E
        pltpu.sync_copy(src_hbm.at[pl.ds(base, CH_E)], sv)
        pltpu.sync_copy(dst_hbm.at[pl.ds(base, CH_E)], dv)
        pltpu.async_copy(dis_hbm.at[sv], av, sem).wait()
        pltpu.async_copy(dis_hbm.at[dv], bv, sem).wait()

        @pl.loop(0, CH_E)
        def _(e):
            nv[e] = -(av[e, pl.ds(0, 16)] * bv[e, pl.ds(0, 16)])

        pltpu.sync_copy(nv, out_hbm.at[pl.ds(base, CH_E)])


NCH = EPT // CH_E


def _make_lap_kernel(T):
    @functools.partial(
        pl.kernel,
        out_type=jax.ShapeDtypeStruct((T, NP_, 256), jnp.float32),
        mesh=_mesh,
        scratch_types=[
            pltpu.VMEM((2, CH_E), jnp.int32),
            pltpu.VMEM((2, CH_E), jnp.int32),
            pltpu.VMEM((2, CH_E * 16), jnp.float32),
            pltpu.VMEM((2, CH_E, 128), jnp.float32),
            pltpu.VMEM((ROWS_T // 8, 128), jnp.float32),
            pltpu.VMEM_SHARED((NP_, 128), jnp.float32),
        ] + [pltpu.SemaphoreType.DMA] * 8,
    )
    def lap_kernel(tab_hbm, gidx_hbm, sidx_hbm, nrm_hbm, out_hbm,
                   gi_c, si_c, nm_c, rows, zbuf, acc,
                   sg0, sg1, sgi0, sgi1, ssn0, ssn1, ssc0, ssc1):
        c = lax.axis_index("c")
        s = lax.axis_index("s")
        zero16 = jnp.zeros((16,), jnp.float32)
        sg = (sg0, sg1)
        sgi = (sgi0, sgi1)
        ssn = (ssn0, ssn1)
        ssc = (ssc0, ssc1)

        @pl.loop(0, ROWS_T // 8)
        def _(i):
            for j in range(8):
                zbuf[i, pl.ds(j * 16, 16)] = zero16

        def zero_slab():
            for q in range(8):
                pltpu.sync_copy(
                    zbuf,
                    acc.at[pl.ds(s * ROWS_T + q * (ROWS_T // 8),
                                 ROWS_T // 8)])

        zero_slab()
        plsc.subcore_barrier()

        @pl.loop(0, T)
        def _(t):
            def gi_src(k):
                base = s * EPT + k * CH_E
                return gidx_hbm.at[t, c, pl.ds(base, CH_E)]

            def si_src(k):
                base = s * EPT + k * CH_E
                return sidx_hbm.at[pl.ds(base, CH_E)]

            def nm_src(k):
                base = s * EPT + k * CH_E
                return nrm_hbm.at[pl.ds(base * 16, CH_E * 16)]

            def fire_gi(k, b):
                pltpu.async_copy(gi_src(k), gi_c.at[b], sgi[b])

            def wait_gi(k, b):
                pltpu.make_async_copy(gi_src(k), gi_c.at[b], sgi[b]).wait()

            def fire_sn(k, b):
                pltpu.async_copy(si_src(k), si_c.at[b], ssn[b])
                pltpu.async_copy(nm_src(k), nm_c.at[b], ssn[b])

            def wait_sn(k, b):
                pltpu.make_async_copy(si_src(k), si_c.at[b], ssn[b]).wait()
                pltpu.make_async_copy(nm_src(k), nm_c.at[b], ssn[b]).wait()

            def fire_gather(b):
                pltpu.async_copy(tab_hbm.at[gi_c.at[b]], rows.at[b], sg[b])

            def wait_gather(b):
                pltpu.make_async_copy(
                    tab_hbm.at[gi_c.at[b]], rows.at[b], sg[b]).wait()

            def fire_scatter(b):
                pltpu.async_copy(rows.at[b], acc.at[si_c.at[b]], ssc[b],
                                 add=True)

            def wait_scatter(b):
                pltpu.make_async_copy(rows.at[b], acc.at[si_c.at[b]],
                                      ssc[b]).wait()

            pltpu.sync_copy(gi_src(0), gi_c.at[0])
            pltpu.sync_copy(si_src(0), si_c.at[0])
            pltpu.sync_copy(nm_src(0), nm_c.at[0])
            fire_gather(0)
            fire_gi(1, 1)

            @pl.loop(0, NCH, step=2)
            def _(k0):
                for b in range(2):
                    k = k0 + b
                    nb = 1 - b

                    wait_gather(b)

                    @pl.when(k + 1 < NCH)
                    def _():
                        wait_gi(k + 1, nb)

                    @pl.when(k >= 1)
                    def _():
                        wait_scatter(nb)

                    @pl.when(k + 1 < NCH)
                    def _():
                        fire_gather(nb)

                    @pl.when(k + 2 < NCH)
                    def _():
                        fire_gi(k + 2, b)

                    @pl.when(k + 1 < NCH)
                    def _():
                        fire_sn(k + 1, nb)

                    @pl.when(k >= 1)
                    def _():
                        wait_sn(k, b)

                    @pl.loop(0, CH_E, unroll=4)
                    def _(e):
                        nrow = nm_c[b, pl.ds(e * 16, 16)]
                        for j in range(8):
                            rows[b, e, pl.ds(j * 16, 16)] = (
                                rows[b, e, pl.ds(j * 16, 16)] * nrow)

                    fire_scatter(b)

            wait_scatter((NCH - 1) % 2)
            plsc.subcore_barrier()
            pltpu.sync_copy(acc.at[pl.ds(s * ROWS_T, ROWS_T)],
                            out_hbm.at[t, pl.ds(s * ROWS_T, ROWS_T),
                                       pl.ds(c * 128, 128)])
            zero_slab()
            plsc.subcore_barrier()

    return lap_kernel


_lap12 = _make_lap_kernel(SEQ)
_lap1 = _make_lap_kernel(1)


def _dis_body(deg_ref, o_ref):
    d = jnp.sum(deg_ref[...], axis=(0, 2)) * (1.0 / 128.0)
    v = jnp.where(d > 0, 1.0 / jnp.sqrt(jnp.maximum(d, 1e-12)), 0.0)
    o_ref[...] = jnp.broadcast_to(v[:, None], (NP_, 128))


_BN = 1000


def _gates_body(t_ref, x_ref, lx1_ref, lx2_ref, h_ref, lh_ref, llh_ref,
                c_ref, w_ref, b_ref, wc_ref, hn_ref, cn_ref):
    X = jnp.concatenate(
        [x_ref[0], lx1_ref[0], lx2_ref[0], h_ref[...], lh_ref[0],
         llh_ref[0]], axis=-1)
    G = lax.dot_general(X, w_ref[...], (((1,), (0,)), ((), ())),
                        preferred_element_type=jnp.float32) + b_ref[...]
    cb = c_ref[...]
    gi = jax.nn.sigmoid(G[:, 0:256] + wc_ref[0:1, :] * cb)
    gf = jax.nn.sigmoid(G[:, 256:512] + wc_ref[1:2, :] * cb)
    gt = jnp.tanh(G[:, 512:768])
    cn = gf * cb + gi * gt
    go = jax.nn.sigmoid(G[:, 768:1024] + wc_ref[2:3, :] * cn)
    hn_ref[...] = go * jnp.tanh(cn)
    cn_ref[...] = cn


_gates_call = pl.pallas_call(
    _gates_body,
    grid_spec=pltpu.PrefetchScalarGridSpec(
        num_scalar_prefetch=1,
        grid=(N // _BN,),
        in_specs=[
            pl.BlockSpec((1, _BN, 256), lambda i, t: (t[0], i, 0)),
            pl.BlockSpec((1, _BN, 256), lambda i, t: (t[0], i, 0)),
            pl.BlockSpec((1, _BN, 256), lambda i, t: (t[0], i, 0)),
            pl.BlockSpec((_BN, 256), lambda i, t: (i, 0)),
            pl.BlockSpec((1, _BN, 256), lambda i, t: (0, i, 0)),
            pl.BlockSpec((1, _BN, 256), lambda i, t: (0, i, 0)),
            pl.BlockSpec((_BN, 256), lambda i, t: (i, 0)),
            pl.BlockSpec((1536, 1024), lambda i, t: (0, 0)),
            pl.BlockSpec((1, 1024), lambda i, t: (0, 0)),
            pl.BlockSpec((3, 256), lambda i, t: (0, 0)),
        ],
        out_specs=[
            pl.BlockSpec((_BN, 256), lambda i, t: (i, 0)),
            pl.BlockSpec((_BN, 256), lambda i, t: (i, 0)),
        ],
    ),
    out_shape=[
        jax.ShapeDtypeStruct((N, 256), jnp.float32),
        jax.ShapeDtypeStruct((N, 256), jnp.float32),
    ],
)


def _head_body(*refs):
    h_refs = refs[:SEQ]
    lng_ref, lnb_ref, w1_ref, b1_ref, w2_ref, b2_ref, o_ref = refs[SEQ:]
    ys = []
    for t in range(SEQ):
        x = jnp.tanh(h_refs[t][...])
        mu = jnp.mean(x, axis=-1, keepdims=True)
        xc = x - mu
        var = jnp.mean(xc * xc, axis=-1, keepdims=True)
        xn = xc / jnp.sqrt(var + 1e-5) * lng_ref[...] + lnb_ref[...]
        y = jnp.maximum(
            lax.dot_general(xn, w1_ref[...], (((1,), (0,)), ((), ())),
                            preferred_element_type=jnp.float32)
            + b1_ref[...], 0.0)
        y = lax.dot_general(y, w2_ref[...], (((1,), (0,)), ((), ())),
                            preferred_element_type=jnp.float32) + b2_ref[...]
        ys.append(jax.nn.sigmoid(y)[:, None, :])
    o_ref[...] = jnp.concatenate(ys, axis=1)


_HBN = 1000
_head_call = pl.pallas_call(
    _head_body,
    grid=(N // _HBN,),
    in_specs=(
        [pl.BlockSpec((_HBN, 256), lambda i: (i, 0)) for _ in range(SEQ)]
        + [
            pl.BlockSpec((1, 256), lambda i: (0, 0)),
            pl.BlockSpec((1, 256), lambda i: (0, 0)),
            pl.BlockSpec((256, HID), lambda i: (0, 0)),
            pl.BlockSpec((1, HID), lambda i: (0, 0)),
            pl.BlockSpec((HID, CH), lambda i: (0, 0)),
            pl.BlockSpec((1, CH), lambda i: (0, 0)),
        ]
    ),
    out_specs=pl.BlockSpec((_HBN, SEQ, CH), lambda i: (i, 0, 0)),
    out_shape=jax.ShapeDtypeStruct((N, SEQ, CH), jnp.float32),
)


def _stack_w(W):
    blocks = jnp.concatenate([W[:, 0] - W[:, 2], W[:, 1], 2.0 * W[:, 2]],
                             axis=1)
    return jnp.moveaxis(blocks, 0, 1).reshape(768, 1024)


def kernel(H, edge_index, Wx, bx, Wh, bh, wc, bg, ln_g, ln_b, W1, b1, W2, b2):
    src = edge_index[0]
    dst = edge_index[1]
    pad = EP - E
    zpad = jnp.zeros((pad,), jnp.int32)
    tpad = jnp.full((pad,), N, jnp.int32)
    srcp0 = jnp.concatenate([src, zpad])
    dstp0 = jnp.concatenate([dst, zpad])
    sidx = jnp.concatenate([dst, tpad])
    degidx = jnp.concatenate([src, tpad])

    hfo = jnp.arange(2, dtype=jnp.int32)[None, :, None]
    t_ar = jnp.arange(SEQ, dtype=jnp.int32)[:, None, None]
    base2 = (2 * srcp0)[None, None, :]
    gidxH = (2 * N) * t_ar + hfo + base2
    gidxL = (2 * NP_) * t_ar + hfo + base2
    idx0 = gidxH[:1]

    Wbig = jnp.concatenate([_stack_w(Wx), _stack_w(Wh)], axis=0)
    bbig = (bx + bh + bg).reshape(1, 1024)
    wc2 = wc

    degparts = _deg_kernel(degidx)
    dis = pl.pallas_call(
        _dis_body,
        out_shape=jax.ShapeDtypeStruct((NP_, 128), jnp.float32),
    )(degparts)
    norm = _norm_kernel(dis, srcp0, dstp0).reshape(EP * 16)

    Htab = H.reshape(SEQ * N * 2, 128)
    LX1 = _lap12(Htab, gidxH, sidx, norm)
    LX2 = _lap12(LX1.reshape(SEQ * NP_ * 2, 128), gidxL, sidx, norm)
    LX1r = LX1.reshape(SEQ, NP_, 256)
    LX2r = LX2.reshape(SEQ, NP_, 256)

    h = jnp.zeros((N, 256), jnp.float32)
    c = jnp.zeros((N, 256), jnp.float32)
    zlap = jnp.zeros((1, NP_, 256), jnp.float32)
    hs = []
    for t in range(SEQ):
        if t == 0:
            lh, llh = zlap, zlap
        else:
            lh4 = _lap1(h.reshape(N * 2, 128), idx0, sidx, norm)
            llh4 = _lap1(lh4.reshape(NP_ * 2, 128), gidxL[:1], sidx, norm)
            lh = lh4.reshape(1, NP_, 256)
            llh = llh4.reshape(1, NP_, 256)
        tt = jnp.full((1,), t, jnp.int32)
        h, c = _gates_call(tt, H, LX1r, LX2r, h, lh, llh, c, Wbig, bbig, wc2)
        hs.append(h)

    out = _head_call(*hs, ln_g.reshape(1, 256), ln_b.reshape(1, 256),
                     W1, b1.reshape(1, HID), W2, b2.reshape(1, CH))
    return jnp.swapaxes(out, 1, 2)


# speedup vs baseline: 3.0193x; 1.0632x over previous
import functools

import jax
import jax.numpy as jnp
from jax import lax
from jax.experimental import pallas as pl
from jax.experimental.pallas import tpu as pltpu
from jax.experimental.pallas import tpu_sc as plsc

N = 10000
E = 160000
SEQ = 12
LIN = 256
CONV = 256
CH = 128
HID = (CONV + CH) // 2

NP_ = 10112
EP = 163840
CH_E = 128
EPT = EP // 16
EPW = EP // 32
ROWS_T = NP_ // 16
NCH = EPT // CH_E
QG = 4
NG = NCH // QG

_mesh = plsc.VectorSubcoreMesh(core_axis_name="c", subcore_axis_name="s")


@functools.partial(
    pl.kernel,
    out_type=jax.ShapeDtypeStruct((2, NP_, 128), jnp.float32),
    mesh=_mesh,
    scratch_types=[
        pltpu.VMEM((CH_E,), jnp.int32),
        pltpu.VMEM((CH_E, 128), jnp.float32),
        pltpu.VMEM((ROWS_T // 8, 128), jnp.float32),
        pltpu.VMEM_SHARED((NP_, 128), jnp.float32),
    ],
)
def _deg_kernel(degidx_hbm, out_hbm, idx_v, ones_v, zbuf, acc):
    c = lax.axis_index("c")
    s = lax.axis_index("s")
    w = c * 16 + s
    one16 = jnp.ones((16,), jnp.float32)
    zero16 = jnp.zeros((16,), jnp.float32)

    @pl.loop(0, ROWS_T // 8)
    def _(i):
        for j in range(8):
            zbuf[i, pl.ds(j * 16, 16)] = zero16

    @pl.loop(0, CH_E)
    def _(i):
        for j in range(8):
            ones_v[i, pl.ds(j * 16, 16)] = one16

    for q in range(8):
        pltpu.sync_copy(
            zbuf,
            acc.at[pl.ds(s * ROWS_T + q * (ROWS_T // 8), ROWS_T // 8)])
    plsc.subcore_barrier()

    @pl.loop(0, EPW // CH_E)
    def _(k):
        base = w * EPW + k * CH_E
        pltpu.sync_copy(degidx_hbm.at[pl.ds(base, CH_E)], idx_v)
        pltpu.sync_copy(ones_v, acc.at[idx_v], add=True)

    plsc.subcore_barrier()
    pltpu.sync_copy(acc.at[pl.ds(s * ROWS_T, ROWS_T)],
                    out_hbm.at[c, pl.ds(s * ROWS_T, ROWS_T)])


def _make_lap_kernel(T):
    @functools.partial(
        pl.kernel,
        out_type=jax.ShapeDtypeStruct((T, NP_, 256), jnp.float32),
        mesh=_mesh,
        scratch_types=[
            pltpu.VMEM((2, QG, CH_E), jnp.int32),
            pltpu.VMEM((2, QG, CH_E), jnp.int32),
            pltpu.VMEM((2, CH_E, 128), jnp.float32),
            pltpu.VMEM((ROWS_T // 8, 128), jnp.float32),
            pltpu.VMEM_SHARED((NP_, 128), jnp.float32),
        ] + [pltpu.SemaphoreType.DMA] * 6,
    )
    def lap_kernel(tab_hbm, gidx_hbm, sidx_hbm, out_hbm,
                   gi_c, si_c, rows, zbuf, acc,
                   sg0, sg1, sa0, sa1, ssc0, ssc1):
        c = lax.axis_index("c")
        s = lax.axis_index("s")
        zero16 = jnp.zeros((16,), jnp.float32)
        sg = (sg0, sg1)
        sa = (sa0, sa1)
        ssc = (ssc0, ssc1)

        @pl.loop(0, ROWS_T // 8)
        def _(i):
            for j in range(8):
                zbuf[i, pl.ds(j * 16, 16)] = zero16

        def zero_slab():
            for q in range(8):
                pltpu.sync_copy(
                    zbuf,
                    acc.at[pl.ds(s * ROWS_T + q * (ROWS_T // 8),
                                 ROWS_T // 8)])

        zero_slab()
        plsc.subcore_barrier()

        @pl.loop(0, T)
        def _(t):
            def aux_srcs(g):
                gidx = s * NG + g
                return (gidx_hbm.at[t, c, gidx], sidx_hbm.at[gidx])

            def fire_aux(g, a):
                g_src, s_src = aux_srcs(g)
                pltpu.async_copy(g_src, gi_c.at[a], sa[a])
                pltpu.async_copy(s_src, si_c.at[a], sa[a])

            def wait_aux(g, a):
                g_src, s_src = aux_srcs(g)
                pltpu.make_async_copy(g_src, gi_c.at[a], sa[a]).wait()
                pltpu.make_async_copy(s_src, si_c.at[a], sa[a]).wait()

            def fire_gather(b, a, q):
                pltpu.async_copy(tab_hbm.at[gi_c.at[a, q]], rows.at[b],
                                 sg[b])

            def wait_gather(b, a, q):
                pltpu.make_async_copy(tab_hbm.at[gi_c.at[a, q]],
                                      rows.at[b], sg[b]).wait()

            def fire_scatter(b, a, q):
                pltpu.async_copy(rows.at[b], acc.at[si_c.at[a, q]],
                                 ssc[b], add=True)

            def wait_scatter(b, a, q):
                pltpu.make_async_copy(rows.at[b], acc.at[si_c.at[a, q]],
                                      ssc[b]).wait()

            g_src, s_src = aux_srcs(0)
            pltpu.sync_copy(g_src, gi_c.at[0])
            pltpu.sync_copy(s_src, si_c.at[0])
            fire_gather(0, 0, 0)

            @pl.loop(0, NG, step=2)
            def _(g0):
                for gg in range(2):
                    g = g0 + gg
                    ga = gg
                    nga = 1 - gg
                    for q in range(QG):
                        k = g * QG + q
                        b = q % 2
                        nb = 1 - b
                        pa, pq = (ga, q - 1) if q > 0 else (nga, QG - 1)
                        na, nq = (ga, q + 1) if q < QG - 1 else (nga, 0)

                        wait_gather(b, ga, q)

                        @pl.when(k >= 1)
                        def _():
                            wait_scatter(nb, pa, pq)

                        if q == 1:
                            @pl.when(g + 1 < NG)
                            def _():
                                fire_aux(g + 1, nga)
                        if q == QG - 1:
                            @pl.when(g + 1 < NG)
                            def _():
                                wait_aux(g + 1, nga)

                        @pl.when(k + 1 < NCH)
                        def _():
                            fire_gather(nb, na, nq)

                        fire_scatter(b, ga, q)

            wait_scatter((NCH - 1) % 2, (NG - 1) % 2, QG - 1)
            plsc.subcore_barrier()
            pltpu.sync_copy(acc.at[pl.ds(s * ROWS_T, ROWS_T)],
                            out_hbm.at[t, pl.ds(s * ROWS_T, ROWS_T),
                                       pl.ds(c * 128, 128)])
            zero_slab()
            plsc.subcore_barrier()

    return lap_kernel


_lap12 = _make_lap_kernel(SEQ)
_lap1 = _make_lap_kernel(1)


def _dis_body(deg_ref, o_ref):
    d = jnp.sum(deg_ref[...], axis=(0, 2)) * (1.0 / 128.0)
    v = jnp.where(d > 0, 1.0 / jnp.sqrt(jnp.maximum(d, 1e-12)), 0.0)
    o_ref[...] = jnp.broadcast_to(v[:, None], (NP_, 128))


def _scale_x_body(x_ref, d_ref, o_ref):
    o_ref[...] = x_ref[...] * d_ref[:, 0][None, :, None]


_scale_x = pl.pallas_call(
    _scale_x_body,
    grid=(SEQ, N // 1000),
    in_specs=[
        pl.BlockSpec((1, 1000, 256), lambda t, i: (t, i, 0)),
        pl.BlockSpec((1000, 128), lambda t, i: (i, 0)),
    ],
    out_specs=pl.BlockSpec((1, 1000, 256), lambda t, i: (t, i, 0)),
    out_shape=jax.ShapeDtypeStruct((SEQ, N, 256), jnp.float32),
)


def _scale_mid_body(x_ref, d_ref, o_ref):
    d = d_ref[:, 0][None, :, None]
    o_ref[...] = x_ref[...] * (-(d * d))


def _make_scale_mid(T):
    bn = 632
    return pl.pallas_call(
        _scale_mid_body,
        grid=(T, NP_ // bn),
        in_specs=[
            pl.BlockSpec((1, bn, 256), lambda t, i: (t, i, 0)),
            pl.BlockSpec((bn, 128), lambda t, i: (i, 0)),
        ],
        out_specs=pl.BlockSpec((1, bn, 256), lambda t, i: (t, i, 0)),
        out_shape=jax.ShapeDtypeStruct((T, NP_, 256), jnp.float32),
    )


_scale_mid12 = _make_scale_mid(SEQ)
_scale_mid1 = _make_scale_mid(1)


_BN = 1000


def _gates_body(t_ref, x_ref, lx1_ref, lx2_ref, h_ref, lh_ref, llh_ref,
                c_ref, d_ref, w_ref, b_ref, wc_ref,
                hn_ref, cn_ref, hs_ref):
    d = d_ref[:, 0][:, None]
    nd = -d
    X = jnp.concatenate(
        [x_ref[0], nd * lx1_ref[0], nd * lx2_ref[0], h_ref[...],
         nd * lh_ref[0], nd * llh_ref[0]], axis=-1)
    G = lax.dot_general(X, w_ref[...], (((1,), (0,)), ((), ())),
                        preferred_element_type=jnp.float32) + b_ref[...]
    cb = c_ref[...]
    gi = jax.nn.sigmoid(G[:, 0:256] + wc_ref[0:1, :] * cb)
    gf = jax.nn.sigmoid(G[:, 256:512] + wc_ref[1:2, :] * cb)
    gt = jnp.tanh(G[:, 512:768])
    cn = gf * cb + gi * gt
    go = jax.nn.sigmoid(G[:, 768:1024] + wc_ref[2:3, :] * cn)
    hn = go * jnp.tanh(cn)
    hn_ref[...] = hn
    cn_ref[...] = cn
    hs_ref[...] = d * hn


_gates_call = pl.pallas_call(
    _gates_body,
    grid_spec=pltpu.PrefetchScalarGridSpec(
        num_scalar_prefetch=1,
        grid=(N // _BN,),
        in_specs=[
            pl.BlockSpec((1, _BN, 256), lambda i, t: (t[0], i, 0)),
            pl.BlockSpec((1, _BN, 256), lambda i, t: (t[0], i, 0)),
            pl.BlockSpec((1, _BN, 256), lambda i, t: (t[0], i, 0)),
            pl.BlockSpec((_BN, 256), lambda i, t: (i, 0)),
            pl.BlockSpec((1, _BN, 256), lambda i, t: (0, i, 0)),
            pl.BlockSpec((1, _BN, 256), lambda i, t: (0, i, 0)),
            pl.BlockSpec((_BN, 256), lambda i, t: (i, 0)),
            pl.BlockSpec((_BN, 128), lambda i, t: (i, 0)),
            pl.BlockSpec((1536, 1024), lambda i, t: (0, 0)),
            pl.BlockSpec((1, 1024), lambda i, t: (0, 0)),
            pl.BlockSpec((3, 256), lambda i, t: (0, 0)),
        ],
        out_specs=[
            pl.BlockSpec((_BN, 256), lambda i, t: (i, 0)),
            pl.BlockSpec((_BN, 256), lambda i, t: (i, 0)),
            pl.BlockSpec((_BN, 256), lambda i, t: (i, 0)),
        ],
    ),
    out_shape=[
        jax.ShapeDtypeStruct((N, 256), jnp.float32),
        jax.ShapeDtypeStruct((N, 256), jnp.float32),
        jax.ShapeDtypeStruct((N, 256), jnp.float32),
    ],
)


def _head_body(*refs):
    h_refs = refs[:SEQ]
    lng_ref, lnb_ref, w1_ref, b1_ref, w2_ref, b2_ref, o_ref = refs[SEQ:]
    ys = []
    for t in range(SEQ):
        x = jnp.tanh(h_refs[t][...])
        mu = jnp.mean(x, axis=-1, keepdims=True)
        xc = x - mu
        var = jnp.mean(xc * xc, axis=-1, keepdims=True)
        xn = xc / jnp.sqrt(var + 1e-5) * lng_ref[...] + lnb_ref[...]
        y = jnp.maximum(
            lax.dot_general(xn, w1_ref[...], (((1,), (0,)), ((), ())),
                            preferred_element_type=jnp.float32)
            + b1_ref[...], 0.0)
        y = lax.dot_general(y, w2_ref[...], (((1,), (0,)), ((), ())),
                            preferred_element_type=jnp.float32) + b2_ref[...]
        ys.append(jax.nn.sigmoid(y)[:, None, :])
    o_ref[...] = jnp.concatenate(ys, axis=1)


_HBN = 1000
_head_call = pl.pallas_call(
    _head_body,
    grid=(N // _HBN,),
    in_specs=(
        [pl.BlockSpec((_HBN, 256), lambda i: (i, 0)) for _ in range(SEQ)]
        + [
            pl.BlockSpec((1, 256), lambda i: (0, 0)),
            pl.BlockSpec((1, 256), lambda i: (0, 0)),
            pl.BlockSpec((256, HID), lambda i: (0, 0)),
            pl.BlockSpec((1, HID), lambda i: (0, 0)),
            pl.BlockSpec((HID, CH), lambda i: (0, 0)),
            pl.BlockSpec((1, CH), lambda i: (0, 0)),
        ]
    ),
    out_specs=pl.BlockSpec((_HBN, SEQ, CH), lambda i: (i, 0, 0)),
    out_shape=jax.ShapeDtypeStruct((N, SEQ, CH), jnp.float32),
)


def _stack_w(W):
    blocks = jnp.concatenate([W[:, 0] - W[:, 2], W[:, 1], 2.0 * W[:, 2]],
                             axis=1)
    return jnp.moveaxis(blocks, 0, 1).reshape(768, 1024)


def kernel(H, edge_index, Wx, bx, Wh, bh, wc, bg, ln_g, ln_b, W1, b1, W2, b2):
    src = edge_index[0]
    dst = edge_index[1]
    pad = EP - E
    zpad = jnp.zeros((pad,), jnp.int32)
    tpad = jnp.full((pad,), N, jnp.int32)
    srcp0 = jnp.concatenate([src, zpad])
    sidx = jnp.concatenate([dst, tpad])
    degidx = jnp.concatenate([src, tpad])

    hfo = jnp.arange(2, dtype=jnp.int32)[None, :, None]
    t_ar = jnp.arange(SEQ, dtype=jnp.int32)[:, None, None]
    base2 = (2 * srcp0)[None, None, :]
    gidxH = (2 * N) * t_ar + hfo + base2
    gidxL = (2 * NP_) * t_ar + hfo + base2

    Wbig = jnp.concatenate([_stack_w(Wx), _stack_w(Wh)], axis=0)
    bbig = (bx + bh + bg).reshape(1, 1024)

    degparts = _deg_kernel(degidx)
    dis = pl.pallas_call(
        _dis_body,
        out_shape=jax.ShapeDtypeStruct((NP_, 128), jnp.float32),
    )(degparts)

    Xs = _scale_x(H, dis)
    gidxHg = gidxH.reshape(SEQ, 2, EP // (QG * CH_E), QG, CH_E)
    gidxLg = gidxL.reshape(SEQ, 2, EP // (QG * CH_E), QG, CH_E)
    sidxg = sidx.reshape(EP // (QG * CH_E), QG, CH_E)
    P1 = _lap12(Xs.reshape(SEQ * N * 2, 128), gidxHg, sidxg)
    M1 = _scale_mid12(P1, dis)
    P2 = _lap12(M1.reshape(SEQ * NP_ * 2, 128), gidxLg, sidxg)

    h = jnp.zeros((N, 256), jnp.float32)
    c = jnp.zeros((N, 256), jnp.float32)
    hsc = jnp.zeros((N, 256), jnp.float32)
    zlap = jnp.zeros((1, NP_, 256), jnp.float32)
    hs = []
    for t in range(SEQ):
        if t == 0:
            lh, llh = zlap, zlap
        else:
            p1h = _lap1(hsc.reshape(N * 2, 128), gidxHg[:1], sidxg)
            m1h = _scale_mid1(p1h, dis)
            p2h = _lap1(m1h.reshape(NP_ * 2, 128), gidxLg[:1], sidxg)
            lh, llh = p1h, p2h
        tt = jnp.full((1,), t, jnp.int32)
        h, c, hsc = _gates_call(tt, H, P1, P2, h, lh, llh, c, dis,
                                Wbig, bbig, wc)
        hs.append(h)

    out = _head_call(*hs, ln_g.reshape(1, 256), ln_b.reshape(1, 256),
                     W1, b1.reshape(1, HID), W2, b2.reshape(1, CH))
    return jnp.swapaxes(out, 1, 2)
